# baseline (speedup 1.0000x reference)
import jax
import jax.numpy as jnp
from jax.experimental import pallas as pl

_F32 = jnp.float32
_BS = 16


def _bmm(a, b):
    return jax.lax.dot_general(a, b, (((2,), (0,)), ((), ())),
                               preferred_element_type=_F32)


def _tcg_step(cur, Wp, Wc, Wo, x2, topk, n, d):
    t = _bmm(cur, Wp)
    if x2 is not None:
        t = t + x2
    bs = cur.shape[0]
    sim = jax.lax.dot_general(t, t, (((2,), (2,)), ((0,), (0,))),
                              preferred_element_type=_F32)
    colf = jax.lax.broadcasted_iota(jnp.int32, (bs, n, n), 2).astype(_F32)
    work = sim
    msel = None
    for j in range(topk):
        rowmax = jnp.max(work, axis=2, keepdims=True)
        first = jnp.min(jnp.where(work == rowmax, colf, float(n)),
                        axis=2, keepdims=True)
        eq = colf == first
        tsel = jax.lax.dot_general(eq.astype(_F32), t,
                                   (((2,), (1,)), ((0,), (0,))),
                                   preferred_element_type=_F32)
        msel = tsel if msel is None else jnp.maximum(msel, tsel)
        if j < topk - 1:
            work = jnp.where(eq, -jnp.inf, work)
    rel = msel - t
    h = _bmm(t, Wc[:d]) + _bmm(rel, Wc[d:])
    t_out = jax.nn.gelu(h) + t
    pix = _bmm(t_out, Wo)
    return pix, t_out


def _tcg_core(t, Wc, Wo, topk, n, d):
    bs = t.shape[0]
    sim = jax.lax.dot_general(t, t, (((2,), (2,)), ((0,), (0,))),
                              preferred_element_type=_F32)
    colf = jax.lax.broadcasted_iota(jnp.int32, (bs, n, n), 2).astype(_F32)
    work = sim
    msel = None
    for j in range(topk):
        rowmax = jnp.max(work, axis=2, keepdims=True)
        first = jnp.min(jnp.where(work == rowmax, colf, float(n)),
                        axis=2, keepdims=True)
        eq = colf == first
        tsel = jax.lax.dot_general(eq.astype(_F32), t,
                                   (((2,), (1,)), ((0,), (0,))),
                                   preferred_element_type=_F32)
        msel = tsel if msel is None else jnp.maximum(msel, tsel)
        if j < topk - 1:
            work = jnp.where(eq, -jnp.inf, work)
    rel = msel - t
    h = _bmm(t, Wc[:d]) + _bmm(rel, Wc[d:])
    t_out = jax.nn.gelu(h) + t
    pix = _bmm(t_out, Wo)
    return pix, t_out


def _mega_kernel(xr_ref, wexp_ref, d1p_ref, d1c_ref, d1o_ref, s1p_ref,
                 s1c_ref, s1o_ref, d2p_ref, d2c_ref, d2o_ref, s2p_ref,
                 s2c_ref, s2o_ref, w21uv_ref, s21c_ref, s21o_ref, d3p_ref,
                 d3c_ref, d3o_ref, s3p_ref, s3c_ref, s3o_ref, fc_ref,
                 bng_ref, bnb_ref, w1_ref, b1_ref, w2_ref, b2_ref, o_ref):
    bs = xr_ref.shape[0]
    cur = jax.nn.gelu(_bmm(xr_ref[...], wexp_ref[...]))
    cur, x2 = _tcg_step(cur, d1p_ref[...], d1c_ref[...], d1o_ref[...],
                        None, 2, 196, 92)
    for i in range(5):
        cur, x2 = _tcg_step(cur, s1p_ref[i], s1c_ref[i], s1o_ref[i],
                            x2, 2, 196, 92)
    cur, x2 = _tcg_step(cur, d2p_ref[...], d2c_ref[...], d2o_ref[...],
                        None, 2, 196, 192)
    for i in range(2):
        cur, x2 = _tcg_step(cur, s2p_ref[i], s2c_ref[i], s2o_ref[i],
                            x2, 2, 196, 192)
    rowf = jax.lax.broadcasted_iota(jnp.int32, (bs, 49, 196), 1).astype(_F32)
    colq = jax.lax.broadcasted_iota(jnp.int32, (bs, 49, 196), 2).astype(_F32)
    mi = jnp.floor((rowf + 0.5) * (1.0 / 7.0))
    base = 28.0 * mi + 2.0 * (rowf - 7.0 * mi)
    t = None
    for uv in range(4):
        off = 14.0 * (uv // 2) + (uv % 2)
        g = (colq == base + off).astype(_F32)
        sel = jax.lax.dot_general(g, cur, (((2,), (1,)), ((0,), (0,))),
                                  preferred_element_type=_F32)
        contrib = _bmm(sel, w21uv_ref[uv])
        t = contrib if t is None else t + contrib
    cur, x2 = _tcg_core(t, s21c_ref[...], s21o_ref[...], 2, 49, 192)
    cur, x2 = _tcg_step(cur, d3p_ref[...], d3c_ref[...], d3o_ref[...],
                        None, 9, 49, 384)
    for i in range(2):
        cur, x2 = _tcg_step(cur, s3p_ref[i], s3c_ref[i], s3o_ref[i],
                            x2, 2, 49, 384)
    f = _bmm(cur, fc_ref[...])
    f = f * bng_ref[...] + bnb_ref[...]
    f = f * jax.nn.sigmoid(f)
    fm = jnp.mean(f, axis=1)
    h2 = jax.nn.gelu(jnp.dot(fm, w1_ref[...], preferred_element_type=_F32)
                     + b1_ref[...])
    o_ref[...] = jnp.dot(h2, w2_ref[...], preferred_element_type=_F32) + b2_ref[...]


def _full(shape):
    nd = len(shape)
    return pl.BlockSpec(shape, lambda b: (0,) * nd)


def _tile(shape, bs):
    nd = len(shape)
    return pl.BlockSpec((bs,) + tuple(shape[1:]),
                        lambda b: (b,) + (0,) * (nd - 1))


def kernel(x, stem_W, ds1_Wp, ds1_Wc, ds1_Wo, s1_Wp, s1_Wc, s1_Wo,
           ds2_Wp, ds2_Wc, ds2_Wo, s20_Wp, s20_Wc, s20_Wo,
           s21_Wp, s21_Wc, s21_Wo, ds3_Wp, ds3_Wc, ds3_Wo,
           s3_Wp, s3_Wc, s3_Wo, fc_W, bn_g, bn_b,
           head_W1, head_b1, head_W2, head_b2):
    B = x.shape[0]

    xr = x.reshape(B, 3, 14, 8, 14, 8).transpose(0, 2, 4, 1, 3, 5)
    xr = xr.reshape(B, 196, 192)

    w_exp = jnp.einsum('kc,ab->akcb', stem_W, jnp.eye(16, dtype=_F32))
    w_exp = w_exp.reshape(4, 4, 3, 2, 2, 46 * 16)
    w_exp = w_exp.transpose(2, 0, 3, 1, 4, 5).reshape(192, 736)

    w21uv = s21_Wp.reshape(192, 4, 192).transpose(1, 0, 2)

    args = (xr, w_exp, ds1_Wp, ds1_Wc, ds1_Wo, s1_Wp, s1_Wc, s1_Wo,
            ds2_Wp, ds2_Wc, ds2_Wo, s20_Wp, s20_Wc, s20_Wo,
            w21uv, s21_Wc, s21_Wo, ds3_Wp, ds3_Wc, ds3_Wo,
            s3_Wp, s3_Wc, s3_Wo, fc_W, bn_g.reshape(1, 384),
            bn_b.reshape(1, 384), head_W1, head_b1.reshape(1, 1536),
            head_W2, head_b2.reshape(1, 250))
    out = pl.pallas_call(
        _mega_kernel,
        grid=(B // _BS,),
        in_specs=[_tile(xr.shape, _BS)] + [_full(a.shape) for a in args[1:]],
        out_specs=_tile((B, 250), _BS),
        out_shape=jax.ShapeDtypeStruct((B, 250), _F32),
    )(*args)
    return out
